# baseline (speedup 1.0000x reference)
import functools

import jax
import jax.numpy as jnp
from jax import lax
from jax.experimental import pallas as pl
from jax.experimental.pallas import tpu as pltpu
from jax.experimental.pallas import tpu_sc as plsc

_L = 16


def _sc_pe_add(x2, pe):
    BS, D = x2.shape
    S = pe.shape[0]
    NC, NS = 2, 16
    NW = NC * NS
    NB = BS // S
    P = S // NW
    R = 8
    NCI = P // R
    NV = D // _L
    mesh = plsc.VectorSubcoreMesh(core_axis_name="c", subcore_axis_name="s")

    @functools.partial(
        pl.kernel,
        mesh=mesh,
        out_type=jax.ShapeDtypeStruct((BS, D), jnp.float32),
        scratch_types=(
            [pltpu.VMEM((2, R, D), jnp.float32)]
            + [pltpu.VMEM((2, NB, R, D), jnp.float32)]
            + [pltpu.SemaphoreType.DMA] * 2
            + [pltpu.SemaphoreType.DMA] * (2 * NB)
            + [pltpu.SemaphoreType.DMA] * (2 * NB)
        ),
    )
    def k(x_hbm, pe_hbm, out_hbm, pe_v, xb, *sems):
        sem_pe = sems[0:2]
        sem_in = sems[2:2 + 2 * NB]
        sem_out = sems[2 + 2 * NB:2 + 4 * NB]
        wid = lax.axis_index("s") * NC + lax.axis_index("c")
        pe_base = wid * P

        def pe_copy(ci, q):
            return pltpu.make_async_copy(
                pe_hbm.at[pl.ds(pe_base + ci * R, R)], pe_v.at[q], sem_pe[q])

        def in_copy(ci, b, q):
            row = b * S + pe_base + ci * R
            return pltpu.make_async_copy(
                x_hbm.at[pl.ds(row, R)], xb.at[q, b], sem_in[q * NB + b])

        def out_copy(ci, b, q):
            row = b * S + pe_base + ci * R
            return pltpu.make_async_copy(
                xb.at[q, b], out_hbm.at[pl.ds(row, R)], sem_out[q * NB + b])

        pe_copy(0, 0).start()
        for b in range(NB):
            in_copy(0, b, 0).start()

        def group(g, carry):
            for q in (0, 1):
                ci = g * 2 + q
                nq = 1 - q

                @pl.when(ci + 1 < NCI)
                def _issue():
                    pe_copy(ci + 1, nq).start()

                for b in range(NB):
                    @pl.when(ci + 1 < NCI)
                    def _issue_b(b=b):
                        @pl.when(ci >= 1)
                        def _wait_prev_out():
                            out_copy(ci - 1, b, nq).wait()
                        in_copy(ci + 1, b, nq).start()

                pe_copy(ci, q).wait()
                for b in range(NB):
                    in_copy(ci, b, q).wait()

                    def row_body(r, c3, b=b):
                        for j in range(NV):
                            sl = pl.ds(j * _L, _L)
                            plsc.addupdate(xb.at[q, b, r, sl], pe_v[q, r, sl])
                        return c3

                    lax.fori_loop(0, R, row_body, 0, unroll=False)
                    out_copy(ci, b, q).start()
            return carry

        lax.fori_loop(0, NCI // 2, group, 0, unroll=False)

        for b in range(NB):
            out_copy(NCI - 2, b, 0).wait()
            out_copy(NCI - 1, b, 1).wait()

    return k(x2, pe)


def kernel(x, pe):
    B, S, D = x.shape
    x2 = x.reshape(B * S, D)
    out = _sc_pe_add(x2, pe)
    return out.reshape(B, S, D)
